# baseline (speedup 1.0000x reference)
import functools

import jax
import jax.numpy as jnp
from jax import lax
from jax.experimental import pallas as pl
from jax.experimental.pallas import tpu as pltpu
from jax.experimental.pallas import tpu_sc as plsc

NC = 2
NS = 16
NW = NC * NS

CH = 128
NBUF = 4


def _sc_gather(table, idx_flat):
    n = idx_flat.shape[0] * idx_flat.shape[1]
    d = table.shape[1]
    b_per_w = n // NW
    nchunk = b_per_w // CH
    ngroup = nchunk // NBUF
    mesh = plsc.VectorSubcoreMesh(core_axis_name="c", subcore_axis_name="s")

    @functools.partial(
        pl.kernel,
        mesh=mesh,
        compiler_params=pltpu.CompilerParams(use_tc_tiling_on_sc=False),
        out_type=jax.ShapeDtypeStruct((n, d), table.dtype),
        scratch_types=[
            pltpu.VMEM((nchunk, CH), jnp.int32),
            pltpu.VMEM((NBUF, CH, d), table.dtype),
        ]
        + [pltpu.SemaphoreType.DMA] * (2 * NBUF),
    )
    def gather_kernel(table_hbm, idx_hbm, out_hbm, idx_v, rows_v, *sems):
        gsem = sems[:NBUF]
        ssem = sems[NBUF:]
        wid = lax.axis_index("s") * NC + lax.axis_index("c")
        base = wid * b_per_w
        pltpu.sync_copy(idx_hbm.at[pl.ds(wid * nchunk, nchunk)], idx_v)
        for b in range(NBUF):
            pltpu.async_copy(table_hbm.at[idx_v.at[b]], rows_v.at[b], gsem[b])

        def group(t, carry):
            for b in range(NBUF):
                g = t * NBUF + b
                pltpu.make_async_copy(
                    table_hbm.at[idx_v.at[g]], rows_v.at[b], gsem[b]
                ).wait()
                pltpu.async_copy(
                    rows_v.at[b], out_hbm.at[pl.ds(base + g * CH, CH)],
                    ssem[b]).wait()
                pltpu.async_copy(
                    table_hbm.at[idx_v.at[g + NBUF]], rows_v.at[b], gsem[b])
            return carry

        lax.fori_loop(0, ngroup - 1, group, 0)
        for b in range(NBUF):
            g = (ngroup - 1) * NBUF + b
            pltpu.make_async_copy(
                table_hbm.at[idx_v.at[g]], rows_v.at[b], gsem[b]).wait()
            pltpu.async_copy(
                rows_v.at[b], out_hbm.at[pl.ds(base + g * CH, CH)],
                ssem[b]).wait()

    return gather_kernel(table, idx_flat)


def _build_ct(smt, w4, b2):
    def body(smt_ref, w_ref, b_ref, ct_ref):
        for f in range(4):
            blk = jnp.dot(smt_ref[f], w_ref[f],
                          preferred_element_type=jnp.float32)
            if f == 0:
                blk = blk + b_ref[...]
            ct_ref[pl.ds(f * 32, 32), :] = blk.astype(jnp.bfloat16)

    return pl.pallas_call(
        body,
        out_shape=jax.ShapeDtypeStruct((128, 128), jnp.bfloat16),
    )(smt, w4, b2)


def _tc_fuse(rows, xy, xm, xd, xh, ct, pos, gamma2, beta2, tb=16):
    bsz, s, d = rows.shape

    def body(rows_ref, xy_ref, xm_ref, xd_ref, xh_ref, ct_ref, pos_ref,
             g_ref, be_ref, out_ref):
        t = tb * s
        col = lax.broadcasted_iota(jnp.int32, (tb, s, d), 2)
        xyv = xy_ref[...]
        xmv = xm_ref[...] + 32
        xdv = xd_ref[...] + 64
        xhv = xh_ref[...] + 96
        sel = jnp.where(col < 32, xyv,
                        jnp.where(col < 64, xmv,
                                  jnp.where(col < 96, xdv, xhv)))
        oh = (col == sel).astype(jnp.float32)
        ctx = jnp.dot(oh.reshape(t, d).astype(jnp.bfloat16), ct_ref[...],
                      preferred_element_type=jnp.float32)
        emb = ((rows_ref[...].astype(jnp.float32) + pos_ref[...][None])
               + ctx.reshape(tb, s, d))
        emb2 = emb.reshape(t, d)
        one = jnp.ones((d, 1), jnp.float32)
        mean = (jnp.dot(emb2, one, preferred_element_type=jnp.float32)
                * (1.0 / d))
        ex2 = (jnp.dot(emb2 * emb2, one, preferred_element_type=jnp.float32)
               * (1.0 / d))
        rstd = lax.rsqrt(ex2 - mean * mean + 1e-6).reshape(tb, s, 1)
        m3 = mean.reshape(tb, s, 1)
        g3 = g_ref[...][0][None, None, :]
        be3 = be_ref[...][0][None, None, :]
        out_ref[...] = ((emb - m3) * rstd) * g3 + be3

    grid = (bsz // tb,)
    return pl.pallas_call(
        body,
        grid=grid,
        in_specs=[
            pl.BlockSpec((tb, s, d), lambda i: (i, 0, 0)),
            pl.BlockSpec((tb, s, 1), lambda i: (i, 0, 0)),
            pl.BlockSpec((tb, s, 1), lambda i: (i, 0, 0)),
            pl.BlockSpec((tb, s, 1), lambda i: (i, 0, 0)),
            pl.BlockSpec((tb, s, 1), lambda i: (i, 0, 0)),
            pl.BlockSpec((128, 128), lambda i: (0, 0)),
            pl.BlockSpec((s, d), lambda i: (0, 0)),
            pl.BlockSpec((1, d), lambda i: (0, 0)),
            pl.BlockSpec((1, d), lambda i: (0, 0)),
        ],
        out_specs=pl.BlockSpec((tb, s, d), lambda i: (i, 0, 0)),
        out_shape=jax.ShapeDtypeStruct((bsz, s, d), jnp.float32),
    )(rows, xy, xm, xd, xh, ct, pos, gamma2, beta2)


def kernel(x_item, x_year, x_month, x_day, x_hour,
           item_table, year_table, month_table, day_table, hour_table,
           W, b, gamma, beta, pos_table):
    bsz, s = x_item.shape
    d = item_table.shape[1]

    def pad32(t):
        return jnp.zeros((32, t.shape[1]), t.dtype).at[: t.shape[0]].set(t)

    smt = jnp.stack([pad32(year_table), pad32(month_table),
                     pad32(day_table), pad32(hour_table)])
    ct = _build_ct(smt, W.reshape(4, 10, d), b.reshape(1, d))
    table_i32 = lax.bitcast_convert_type(
        item_table.astype(jnp.bfloat16).reshape(-1, d // 2, 2), jnp.int32)
    rows_i32 = _sc_gather(table_i32, x_item.reshape(-1, CH))
    rows = lax.bitcast_convert_type(
        rows_i32.reshape(bsz, s, d // 2), jnp.bfloat16).reshape(bsz, s, d)
    return _tc_fuse(rows,
                    x_year.reshape(bsz, s, 1), x_month.reshape(bsz, s, 1),
                    x_day.reshape(bsz, s, 1), x_hour.reshape(bsz, s, 1),
                    ct, pos_table, gamma.reshape(1, d), beta.reshape(1, d))


# speedup vs baseline: 1.9050x; 1.9050x over previous
import functools

import jax
import jax.numpy as jnp
from jax import lax
from jax.experimental import pallas as pl
from jax.experimental.pallas import tpu as pltpu
from jax.experimental.pallas import tpu_sc as plsc

NC = 2
NS = 16
NW = NC * NS

CH = 128
NBUF = 4


def _sc_gather(table, idx_flat):
    n = idx_flat.shape[0] * idx_flat.shape[1]
    d = table.shape[1]
    b_per_w = n // NW
    nchunk = b_per_w // CH
    ngroup = nchunk // NBUF
    mesh = plsc.VectorSubcoreMesh(core_axis_name="c", subcore_axis_name="s")

    @functools.partial(
        pl.kernel,
        mesh=mesh,
        out_type=jax.ShapeDtypeStruct((n, d), table.dtype),
        scratch_types=[
            pltpu.VMEM((nchunk, CH), jnp.int32),
            pltpu.VMEM((NBUF, CH, d), table.dtype),
        ]
        + [pltpu.SemaphoreType.DMA] * (2 * NBUF),
    )
    def gather_kernel(table_hbm, idx_hbm, out_hbm, idx_v, rows_v, *sems):
        gsem = sems[:NBUF]
        ssem = sems[NBUF:]
        wid = lax.axis_index("s") * NC + lax.axis_index("c")
        base = wid * b_per_w
        pltpu.sync_copy(idx_hbm.at[pl.ds(wid * nchunk, nchunk)], idx_v)
        for b in range(NBUF):
            pltpu.async_copy(table_hbm.at[idx_v.at[b]], rows_v.at[b], gsem[b])

        def group(t, carry):
            for b in range(NBUF):
                g = t * NBUF + b
                pltpu.make_async_copy(
                    table_hbm.at[idx_v.at[g]], rows_v.at[b], gsem[b]
                ).wait()
                pltpu.async_copy(
                    rows_v.at[b], out_hbm.at[pl.ds(base + g * CH, CH)],
                    ssem[b]).wait()
                pltpu.async_copy(
                    table_hbm.at[idx_v.at[g + NBUF]], rows_v.at[b], gsem[b])
            return carry

        lax.fori_loop(0, ngroup - 1, group, 0)
        for b in range(NBUF):
            g = (ngroup - 1) * NBUF + b
            pltpu.make_async_copy(
                table_hbm.at[idx_v.at[g]], rows_v.at[b], gsem[b]).wait()
            pltpu.async_copy(
                rows_v.at[b], out_hbm.at[pl.ds(base + g * CH, CH)],
                ssem[b]).wait()

    return gather_kernel(table, idx_flat)


def _build_ct(smt, w4, b2):
    def body(smt_ref, w_ref, b_ref, ct_ref):
        for f in range(4):
            blk = jnp.dot(smt_ref[f], w_ref[f],
                          preferred_element_type=jnp.float32)
            if f == 0:
                blk = blk + b_ref[...]
            ct_ref[pl.ds(f * 32, 32), :] = blk.astype(jnp.bfloat16)

    return pl.pallas_call(
        body,
        out_shape=jax.ShapeDtypeStruct((128, 128), jnp.bfloat16),
    )(smt, w4, b2)


def _tc_fuse(rows, xy, xm, xd, xh, ct, pos, gamma2, beta2, tb=16):
    bsz, s, d = rows.shape

    def body(rows_ref, xy_ref, xm_ref, xd_ref, xh_ref, ct_ref, pos_ref,
             g_ref, be_ref, out_ref):
        t = tb * s
        col = lax.broadcasted_iota(jnp.int32, (tb, s, d), 2)
        xyv = xy_ref[...]
        xmv = xm_ref[...] + 32
        xdv = xd_ref[...] + 64
        xhv = xh_ref[...] + 96
        sel = jnp.where(col < 32, xyv,
                        jnp.where(col < 64, xmv,
                                  jnp.where(col < 96, xdv, xhv)))
        oh = (col == sel).astype(jnp.float32)
        ctx = jnp.dot(oh.reshape(t, d).astype(jnp.bfloat16), ct_ref[...],
                      preferred_element_type=jnp.float32)
        emb = ((rows_ref[...].astype(jnp.float32) + pos_ref[...][None])
               + ctx.reshape(tb, s, d))
        emb2 = emb.reshape(t, d)
        one = jnp.ones((d, 1), jnp.float32)
        mean = (jnp.dot(emb2, one, preferred_element_type=jnp.float32)
                * (1.0 / d))
        ex2 = (jnp.dot(emb2 * emb2, one, preferred_element_type=jnp.float32)
               * (1.0 / d))
        rstd = lax.rsqrt(ex2 - mean * mean + 1e-6).reshape(tb, s, 1)
        m3 = mean.reshape(tb, s, 1)
        g3 = g_ref[...][0][None, None, :]
        be3 = be_ref[...][0][None, None, :]
        out_ref[...] = ((emb - m3) * rstd) * g3 + be3

    grid = (bsz // tb,)
    return pl.pallas_call(
        body,
        grid=grid,
        in_specs=[
            pl.BlockSpec((tb, s, d), lambda i: (i, 0, 0)),
            pl.BlockSpec((tb, s, 1), lambda i: (i, 0, 0)),
            pl.BlockSpec((tb, s, 1), lambda i: (i, 0, 0)),
            pl.BlockSpec((tb, s, 1), lambda i: (i, 0, 0)),
            pl.BlockSpec((tb, s, 1), lambda i: (i, 0, 0)),
            pl.BlockSpec((128, 128), lambda i: (0, 0)),
            pl.BlockSpec((s, d), lambda i: (0, 0)),
            pl.BlockSpec((1, d), lambda i: (0, 0)),
            pl.BlockSpec((1, d), lambda i: (0, 0)),
        ],
        out_specs=pl.BlockSpec((tb, s, d), lambda i: (i, 0, 0)),
        out_shape=jax.ShapeDtypeStruct((bsz, s, d), jnp.float32),
    )(rows, xy, xm, xd, xh, ct, pos, gamma2, beta2)


def kernel(x_item, x_year, x_month, x_day, x_hour,
           item_table, year_table, month_table, day_table, hour_table,
           W, b, gamma, beta, pos_table):
    bsz, s = x_item.shape
    d = item_table.shape[1]

    def pad32(t):
        return jnp.zeros((32, t.shape[1]), t.dtype).at[: t.shape[0]].set(t)

    smt = jnp.stack([pad32(year_table), pad32(month_table),
                     pad32(day_table), pad32(hour_table)])
    ct = _build_ct(smt, W.reshape(4, 10, d), b.reshape(1, d))
    rows = _sc_gather(item_table, x_item.reshape(-1, CH)).reshape(bsz, s, d)
    return _tc_fuse(rows,
                    x_year.reshape(bsz, s, 1), x_month.reshape(bsz, s, 1),
                    x_day.reshape(bsz, s, 1), x_hour.reshape(bsz, s, 1),
                    ct, pos_table, gamma.reshape(1, d), beta.reshape(1, d))
